# baseline (speedup 1.0000x reference)
import functools

import jax
import jax.numpy as jnp
from jax import lax
from jax.experimental import pallas as pl
from jax.experimental.pallas import tpu as pltpu
from jax.experimental.pallas import tpu_sc as plsc


def _make_pair_gather(B):
    info = plsc.get_sparse_core_info()
    nw = info.num_cores * info.num_subcores
    b_per_w = B // nw
    mesh = plsc.VectorSubcoreMesh(core_axis_name="c", subcore_axis_name="s")

    @functools.partial(
        pl.kernel,
        mesh=mesh,
        out_type=jax.ShapeDtypeStruct((B, 128), jnp.float32),
        scratch_types=[
            pltpu.VMEM((b_per_w,), jnp.int32),
            pltpu.VMEM((b_per_w, 128), jnp.float32),
            pltpu.SemaphoreType.DMA,
        ],
    )
    def gather_kernel(pair_hbm, tbl_hbm, out_hbm, pair_v, rows_v, sem):
        wid = lax.axis_index("s") * info.num_cores + lax.axis_index("c")
        base = wid * b_per_w
        pltpu.sync_copy(pair_hbm.at[pl.ds(base, b_per_w)], pair_v)
        pltpu.async_copy(tbl_hbm.at[pair_v], rows_v, sem).wait()
        pltpu.sync_copy(rows_v, out_hbm.at[pl.ds(base, b_per_w)])

    return gather_kernel


def _select_half(pairs, parity):
    B = pairs.shape[0]
    blk = 512

    def body(par_ref, rows_ref, out_ref):
        par = par_ref[...]
        rows = rows_ref[...]
        out_ref[...] = jnp.where(par == 1, rows[:, 64:128], rows[:, 0:64])

    return pl.pallas_call(
        body,
        out_shape=jax.ShapeDtypeStruct((B, 64), jnp.float32),
        grid=(B // blk,),
        in_specs=[
            pl.BlockSpec((blk, 1), lambda i: (i, 0)),
            pl.BlockSpec((blk, 128), lambda i: (i, 0)),
        ],
        out_specs=pl.BlockSpec((blk, 64), lambda i: (i, 0)),
    )(parity, pairs)


def kernel(target_spk, life_long_mem):
    idx = jnp.reshape(target_spk, (target_spk.shape[0],)).astype(jnp.int32)
    B = idx.shape[0]
    V, D = life_long_mem.shape
    tbl128 = jnp.reshape(life_long_mem, (V // 2, 2 * D))
    pair = idx >> 1
    parity = jnp.reshape(idx & 1, (B, 1))
    pairs = _make_pair_gather(B)(pair, tbl128)
    return _select_half(pairs, parity)


# speedup vs baseline: 1.1666x; 1.1666x over previous
import functools

import jax
import jax.numpy as jnp
from jax import lax
from jax.experimental import pallas as pl
from jax.experimental.pallas import tpu as pltpu
from jax.experimental.pallas import tpu_sc as plsc


def _make_gather(B, V, D):
    info = plsc.get_sparse_core_info()
    nw = info.num_cores * info.num_subcores
    b_per_w = B // nw
    mesh = plsc.VectorSubcoreMesh(core_axis_name="c", subcore_axis_name="s")

    @functools.partial(
        pl.kernel,
        mesh=mesh,
        out_type=jax.ShapeDtypeStruct((B, 2 * D), jnp.float32),
        scratch_types=[
            pltpu.VMEM((b_per_w,), jnp.int32),
            pltpu.VMEM((b_per_w, 2 * D), jnp.float32),
            pltpu.SemaphoreType.DMA,
        ],
    )
    def gather_kernel(idx_hbm, tbl_hbm, out_hbm, idx_v, rows_v, sem):
        wid = lax.axis_index("s") * info.num_cores + lax.axis_index("c")
        base = wid * b_per_w
        pltpu.sync_copy(idx_hbm.at[pl.ds(base, b_per_w)], idx_v)
        pltpu.async_copy(tbl_hbm.at[idx_v], rows_v, sem).wait()
        pltpu.sync_copy(rows_v, out_hbm.at[pl.ds(base, b_per_w)])

    return gather_kernel


def kernel(target_spk, life_long_mem):
    idx = jnp.reshape(target_spk, (target_spk.shape[0],)).astype(jnp.int32)
    B = idx.shape[0]
    V, D = life_long_mem.shape
    tbl_padded = jnp.pad(life_long_mem, ((0, 0), (0, D)))
    return _make_gather(B, V, D)(idx, tbl_padded)[:, :D]
